# baseline (speedup 1.0000x reference)
import functools

import jax
import jax.numpy as jnp
from jax import lax
from jax.experimental import pallas as pl
from jax.experimental.pallas import tpu as pltpu
from jax.experimental.pallas import tpu_sc as plsc

VOCAB = 100000
EMBED_DIM = 64
BATCH = 1024
HIST = 200

_NC = 2
_NS = 16
_NW = _NC * _NS

_B0 = 256
_B1 = BATCH - _B0

_CHUNKS = ((0, 104), (104, 96))


def _make_pool_body(rows_per_w):
    def _pool_body(xf_hbm, emb_hbm, out_hbm, idx_v, rows0_v, rows1_v, out_v,
                   sem0, sem1):
        wid = lax.axis_index("s") * _NC + lax.axis_index("c")
        base = wid * rows_per_w
        pltpu.sync_copy(xf_hbm.at[pl.ds(base * HIST, rows_per_w * HIST)],
                        idx_v)
        bufs = (rows0_v, rows1_v)
        sems = (sem0, sem1)

        def issue(r):
            p = r % 2
            return tuple(
                pltpu.async_copy(
                    emb_hbm.at[idx_v.at[pl.ds(r * HIST + off, ln)]],
                    bufs[p].at[pl.ds(off, ln)],
                    sems[p],
                )
                for off, ln in _CHUNKS
            )

        pend = issue(0)
        inv = 1.0 / HIST
        for r in range(rows_per_w):
            for c in pend:
                c.wait()
            if r + 1 < rows_per_w:
                pend = issue(r + 1)
            buf = bufs[r % 2]

            def acc_body(j, accs, buf=buf):
                j0 = j * 4
                for u in range(4):
                    accs = tuple(
                        accs[k] + buf[j0 + u, pl.ds(16 * k, 16)]
                        for k in range(4)
                    )
                return accs

            z = jnp.zeros((16,), jnp.float32)
            accs = lax.fori_loop(0, HIST // 4, acc_body, (z, z, z, z))
            for k in range(4):
                out_v[r, pl.ds(16 * k, 16)] = accs[k] * inv
        pltpu.sync_copy(out_v, out_hbm.at[pl.ds(base, rows_per_w)])

    return _pool_body


def _pool(xf, emb_table, nrows):
    rows_per_w = nrows // _NW
    mesh = plsc.VectorSubcoreMesh(core_axis_name="c", subcore_axis_name="s")
    fn = pl.kernel(
        _make_pool_body(rows_per_w),
        mesh=mesh,
        out_type=jax.ShapeDtypeStruct((nrows, EMBED_DIM), jnp.float32),
        scratch_types=[
            pltpu.VMEM((rows_per_w * HIST,), jnp.int32),
            pltpu.VMEM((HIST, EMBED_DIM), jnp.float32),
            pltpu.VMEM((HIST, EMBED_DIM), jnp.float32),
            pltpu.VMEM((rows_per_w, EMBED_DIM), jnp.float32),
            pltpu.SemaphoreType.DMA,
            pltpu.SemaphoreType.DMA,
        ],
        compiler_params=pltpu.CompilerParams(use_tc_tiling_on_sc=False),
    )
    return fn(xf, emb_table)


_VB = 4096
_NVB = pl.cdiv(VOCAB, _VB)
_RB = 256


def _mm_body(m_ref, w_ref, b_ref, o_ref):
    o_ref[...] = (
        lax.dot_general(
            m_ref[...],
            w_ref[...],
            dimension_numbers=(((1,), (1,)), ((), ())),
            preferred_element_type=jnp.float32,
        )
        + b_ref[...]
    )


def _mm_body_alias(m_ref, w_ref, b_ref, prev_ref, o_ref):
    del prev_ref
    _mm_body(m_ref, w_ref, b_ref, o_ref)


def _project0(m0, Wb, b2):
    return pl.pallas_call(
        _mm_body,
        grid=(_NVB,),
        in_specs=[
            pl.BlockSpec((_B0, EMBED_DIM), lambda i: (0, 0)),
            pl.BlockSpec((_VB, EMBED_DIM), lambda i: (i, 0)),
            pl.BlockSpec((1, _VB), lambda i: (0, i)),
        ],
        out_specs=pl.BlockSpec((_B0, _VB), lambda i: (0, i)),
        out_shape=jax.ShapeDtypeStruct((BATCH, VOCAB), jnp.float32),
        compiler_params=pltpu.CompilerParams(
            dimension_semantics=("arbitrary",),
        ),
    )(m0, Wb, b2)


def _project1(m1, Wb, b2, prev):
    nrb = _B1 // _RB
    return pl.pallas_call(
        _mm_body_alias,
        grid=(_NVB, nrb),
        in_specs=[
            pl.BlockSpec((_RB, EMBED_DIM), lambda i, r: (r, 0)),
            pl.BlockSpec((_VB, EMBED_DIM), lambda i, r: (i, 0)),
            pl.BlockSpec((1, _VB), lambda i, r: (0, i)),
            pl.BlockSpec(memory_space=pl.ANY),
        ],
        out_specs=pl.BlockSpec((_RB, _VB),
                               lambda i, r: (r + _B0 // _RB, i)),
        out_shape=jax.ShapeDtypeStruct((BATCH, VOCAB), jnp.float32),
        input_output_aliases={3: 0},
        compiler_params=pltpu.CompilerParams(
            dimension_semantics=("arbitrary", "arbitrary"),
        ),
    )(m1, Wb, b2, prev)


def kernel(x, emb_table, W, b):
    xf = x.astype(jnp.int32).reshape(-1)
    Wb = W.astype(jnp.bfloat16)
    b2 = b.reshape(1, VOCAB)
    m0 = _pool(xf[: _B0 * HIST], emb_table, _B0)
    m1 = _pool(xf[_B0 * HIST:], emb_table, _B1)
    out = _project0(m0.astype(jnp.bfloat16), Wb, b2)
    return _project1(m1.astype(jnp.bfloat16), Wb, b2, out)


# speedup vs baseline: 1.0684x; 1.0684x over previous
import functools

import jax
import jax.numpy as jnp
from jax import lax
from jax.experimental import pallas as pl
from jax.experimental.pallas import tpu as pltpu
from jax.experimental.pallas import tpu_sc as plsc

VOCAB = 100000
EMBED_DIM = 64
BATCH = 1024
HIST = 200

_NC = 2
_NS = 16
_NW = _NC * _NS
_ROWS_PER_W = BATCH // _NW

_CHUNKS = ((0, 104), (104, 96))


def _pool_body(xf_hbm, emb_hbm, out_hbm, idx_v, rows0_v, rows1_v, out_v,
               sem0, sem1):
    wid = lax.axis_index("s") * _NC + lax.axis_index("c")
    base = wid * _ROWS_PER_W
    pltpu.sync_copy(xf_hbm.at[pl.ds(base * HIST, _ROWS_PER_W * HIST)], idx_v)
    bufs = (rows0_v, rows1_v)
    sems = (sem0, sem1)

    def issue(r):
        p = r % 2
        return tuple(
            pltpu.async_copy(
                emb_hbm.at[idx_v.at[pl.ds(r * HIST + off, ln)]],
                bufs[p].at[pl.ds(off, ln)],
                sems[p],
            )
            for off, ln in _CHUNKS
        )

    pend = issue(0)
    inv = 1.0 / HIST
    for r in range(_ROWS_PER_W):
        for c in pend:
            c.wait()
        if r + 1 < _ROWS_PER_W:
            pend = issue(r + 1)
        buf = bufs[r % 2]

        def acc_body(j, accs, buf=buf):
            j0 = j * 4
            for u in range(4):
                accs = tuple(
                    accs[k] + buf[j0 + u, pl.ds(16 * k, 16)]
                    for k in range(4)
                )
            return accs

        z = jnp.zeros((16,), jnp.float32)
        accs = lax.fori_loop(0, HIST // 4, acc_body, (z, z, z, z))
        for k in range(4):
            out_v[r, pl.ds(16 * k, 16)] = accs[k] * inv
    pltpu.sync_copy(out_v, out_hbm.at[pl.ds(base, _ROWS_PER_W)])


def _pool(xf, emb_table):
    mesh = plsc.VectorSubcoreMesh(core_axis_name="c", subcore_axis_name="s")
    fn = pl.kernel(
        _pool_body,
        mesh=mesh,
        out_type=jax.ShapeDtypeStruct((BATCH, EMBED_DIM), jnp.float32),
        scratch_types=[
            pltpu.VMEM((_ROWS_PER_W * HIST,), jnp.int32),
            pltpu.VMEM((HIST, EMBED_DIM), jnp.float32),
            pltpu.VMEM((HIST, EMBED_DIM), jnp.float32),
            pltpu.VMEM((_ROWS_PER_W, EMBED_DIM), jnp.float32),
            pltpu.SemaphoreType.DMA,
            pltpu.SemaphoreType.DMA,
        ],
        compiler_params=pltpu.CompilerParams(use_tc_tiling_on_sc=False),
    )
    return fn(xf, emb_table)


_VB = 4096


def _mm_body(m_ref, w_ref, b_ref, o_ref):
    o_ref[...] = (
        lax.dot_general(
            m_ref[...],
            w_ref[...],
            dimension_numbers=(((1,), (1,)), ((), ())),
            precision=lax.Precision.DEFAULT,
            preferred_element_type=jnp.float32,
        )
        + b_ref[...]
    )


def _project(m, W, b2):
    return pl.pallas_call(
        _mm_body,
        grid=(pl.cdiv(VOCAB, _VB),),
        in_specs=[
            pl.BlockSpec((BATCH, EMBED_DIM), lambda i: (0, 0)),
            pl.BlockSpec((_VB, EMBED_DIM), lambda i: (i, 0)),
            pl.BlockSpec((1, _VB), lambda i: (0, i)),
        ],
        out_specs=pl.BlockSpec((BATCH, _VB), lambda i: (0, i)),
        out_shape=jax.ShapeDtypeStruct((BATCH, VOCAB), jnp.float32),
        compiler_params=pltpu.CompilerParams(
            dimension_semantics=("arbitrary",),
        ),
    )(m, W, b2)


def kernel(x, emb_table, W, b):
    xf = x.astype(jnp.int32).reshape(-1)
    m = _pool(xf, emb_table)
    return _project(m, W, b.reshape(1, VOCAB))


# speedup vs baseline: 1.0804x; 1.0112x over previous
import functools

import jax
import jax.numpy as jnp
from jax import lax
from jax.experimental import pallas as pl
from jax.experimental.pallas import tpu as pltpu
from jax.experimental.pallas import tpu_sc as plsc

VOCAB = 100000
EMBED_DIM = 64
BATCH = 1024
HIST = 200

_NC = 2
_NS = 16
_NW = _NC * _NS
_ROWS_PER_W = BATCH // _NW

_CHUNKS = ((0, 104), (104, 96))


def _pool_body(xf_hbm, emb_hbm, out_hbm, idx_v, rows0_v, rows1_v, out_v,
               sem0, sem1):
    wid = lax.axis_index("s") * _NC + lax.axis_index("c")
    base = wid * _ROWS_PER_W
    pltpu.sync_copy(xf_hbm.at[pl.ds(base * HIST, _ROWS_PER_W * HIST)], idx_v)
    bufs = (rows0_v, rows1_v)
    sems = (sem0, sem1)

    def issue(r):
        p = r % 2
        return tuple(
            pltpu.async_copy(
                emb_hbm.at[idx_v.at[pl.ds(r * HIST + off, ln)]],
                bufs[p].at[pl.ds(off, ln)],
                sems[p],
            )
            for off, ln in _CHUNKS
        )

    pend = issue(0)
    inv = 1.0 / HIST
    for r in range(_ROWS_PER_W):
        for c in pend:
            c.wait()
        if r + 1 < _ROWS_PER_W:
            pend = issue(r + 1)
        buf = bufs[r % 2]

        def acc_body(j, accs, buf=buf):
            j0 = j * 4
            for u in range(4):
                accs = tuple(
                    accs[k] + buf[j0 + u, pl.ds(16 * k, 16)]
                    for k in range(4)
                )
            return accs

        z = jnp.zeros((16,), jnp.float32)
        accs = lax.fori_loop(0, HIST // 4, acc_body, (z, z, z, z))
        for k in range(4):
            out_v[r, pl.ds(16 * k, 16)] = accs[k] * inv
    pltpu.sync_copy(out_v, out_hbm.at[pl.ds(base, _ROWS_PER_W)])


def _pool(xf, emb_table):
    mesh = plsc.VectorSubcoreMesh(core_axis_name="c", subcore_axis_name="s")
    fn = pl.kernel(
        _pool_body,
        mesh=mesh,
        out_type=jax.ShapeDtypeStruct((BATCH, EMBED_DIM), jnp.float32),
        scratch_types=[
            pltpu.VMEM((_ROWS_PER_W * HIST,), jnp.int32),
            pltpu.VMEM((HIST, EMBED_DIM), jnp.float32),
            pltpu.VMEM((HIST, EMBED_DIM), jnp.float32),
            pltpu.VMEM((_ROWS_PER_W, EMBED_DIM), jnp.float32),
            pltpu.SemaphoreType.DMA,
            pltpu.SemaphoreType.DMA,
        ],
        compiler_params=pltpu.CompilerParams(use_tc_tiling_on_sc=False),
    )
    return fn(xf, emb_table)


_VB = 8192
_RB = 512


def _mm_body(m_ref, w_ref, b_ref, o_ref):
    o_ref[...] = (
        lax.dot_general(
            m_ref[...],
            w_ref[...],
            dimension_numbers=(((1,), (1,)), ((), ())),
            preferred_element_type=jnp.float32,
        )
        + b_ref[...]
    )


def _project(m, W, b2):
    return pl.pallas_call(
        _mm_body,
        grid=(pl.cdiv(VOCAB, _VB), BATCH // _RB),
        in_specs=[
            pl.BlockSpec((_RB, EMBED_DIM), lambda i, r: (r, 0)),
            pl.BlockSpec((_VB, EMBED_DIM), lambda i, r: (i, 0)),
            pl.BlockSpec((1, _VB), lambda i, r: (0, i)),
        ],
        out_specs=pl.BlockSpec((_RB, _VB), lambda i, r: (r, i)),
        out_shape=jax.ShapeDtypeStruct((BATCH, VOCAB), jnp.float32),
        compiler_params=pltpu.CompilerParams(
            dimension_semantics=("arbitrary", "arbitrary"),
        ),
    )(m, W, b2)


def kernel(x, emb_table, W, b):
    xf = x.astype(jnp.int32).reshape(-1)
    m = _pool(xf, emb_table)
    return _project(m.astype(jnp.bfloat16), W.astype(jnp.bfloat16),
                    b.reshape(1, VOCAB))


# speedup vs baseline: 1.0984x; 1.0167x over previous
import functools

import jax
import jax.numpy as jnp
from jax import lax
from jax.experimental import pallas as pl
from jax.experimental.pallas import tpu as pltpu
from jax.experimental.pallas import tpu_sc as plsc

VOCAB = 100000
EMBED_DIM = 64
BATCH = 1024
HIST = 200

_NC = 2
_NS = 16
_NW = _NC * _NS
_ROWS_PER_W = BATCH // _NW

_CHUNKS = ((0, 104), (104, 96))


_DEPTH = 3


def _pool_body(xf_hbm, emb_hbm, out_hbm, idx_v, rows0_v, rows1_v, rows2_v,
               out_v, sem0, sem1, sem2):
    wid = lax.axis_index("s") * _NC + lax.axis_index("c")
    base = wid * _ROWS_PER_W
    pltpu.sync_copy(xf_hbm.at[pl.ds(base * HIST, _ROWS_PER_W * HIST)], idx_v)
    bufs = (rows0_v, rows1_v, rows2_v)
    sems = (sem0, sem1, sem2)

    def issue(r):
        p = r % _DEPTH
        return tuple(
            pltpu.async_copy(
                emb_hbm.at[idx_v.at[pl.ds(r * HIST + off, ln)]],
                bufs[p].at[pl.ds(off, ln)],
                sems[p],
            )
            for off, ln in _CHUNKS
        )

    pend = [issue(r) for r in range(_DEPTH - 1)]
    inv = 1.0 / HIST
    for r in range(_ROWS_PER_W):
        for c in pend.pop(0):
            c.wait()
        if r + _DEPTH - 1 < _ROWS_PER_W:
            pend.append(issue(r + _DEPTH - 1))
        buf = bufs[r % _DEPTH]

        def acc_body(j, accs, buf=buf):
            j0 = j * 4
            for u in range(4):
                accs = tuple(
                    accs[k] + buf[j0 + u, pl.ds(16 * k, 16)]
                    for k in range(4)
                )
            return accs

        z = jnp.zeros((16,), jnp.float32)
        accs = lax.fori_loop(0, HIST // 4, acc_body, (z, z, z, z))
        for k in range(4):
            out_v[r, pl.ds(16 * k, 16)] = accs[k] * inv
    pltpu.sync_copy(out_v, out_hbm.at[pl.ds(base, _ROWS_PER_W)])


def _pool(xf, emb_table):
    mesh = plsc.VectorSubcoreMesh(core_axis_name="c", subcore_axis_name="s")
    fn = pl.kernel(
        _pool_body,
        mesh=mesh,
        out_type=jax.ShapeDtypeStruct((BATCH, EMBED_DIM), jnp.float32),
        scratch_types=[
            pltpu.VMEM((_ROWS_PER_W * HIST,), jnp.int32),
            pltpu.VMEM((HIST, EMBED_DIM), jnp.float32),
            pltpu.VMEM((HIST, EMBED_DIM), jnp.float32),
            pltpu.VMEM((HIST, EMBED_DIM), jnp.float32),
            pltpu.VMEM((_ROWS_PER_W, EMBED_DIM), jnp.float32),
            pltpu.SemaphoreType.DMA,
            pltpu.SemaphoreType.DMA,
            pltpu.SemaphoreType.DMA,
        ],
        compiler_params=pltpu.CompilerParams(use_tc_tiling_on_sc=False),
    )
    return fn(xf, emb_table)


_VB = 4096
_RB = 1024


def _mm_body(m_ref, w_ref, b_ref, o_ref):
    o_ref[...] = (
        lax.dot_general(
            m_ref[...],
            w_ref[...],
            dimension_numbers=(((1,), (1,)), ((), ())),
            preferred_element_type=jnp.float32,
        )
        + b_ref[...]
    )


def _project(m, W, b2):
    return pl.pallas_call(
        _mm_body,
        grid=(pl.cdiv(VOCAB, _VB), BATCH // _RB),
        in_specs=[
            pl.BlockSpec((_RB, EMBED_DIM), lambda i, r: (r, 0)),
            pl.BlockSpec((_VB, EMBED_DIM), lambda i, r: (i, 0)),
            pl.BlockSpec((1, _VB), lambda i, r: (0, i)),
        ],
        out_specs=pl.BlockSpec((_RB, _VB), lambda i, r: (r, i)),
        out_shape=jax.ShapeDtypeStruct((BATCH, VOCAB), jnp.float32),
        compiler_params=pltpu.CompilerParams(
            dimension_semantics=("arbitrary", "arbitrary"),
        ),
    )(m, W, b2)


def kernel(x, emb_table, W, b):
    xf = x.astype(jnp.int32).reshape(-1)
    m = _pool(xf, emb_table)
    return _project(m.astype(jnp.bfloat16), W.astype(jnp.bfloat16),
                    b.reshape(1, VOCAB))
